# Initial kernel scaffold; baseline (speedup 1.0000x reference)
import jax
import jax.numpy as jnp
from jax.experimental import pallas as pl


def kernel(x, edge_index, Wc, bc, W1, b1, W2, b2, W3, b3):
    raise NotImplementedError("write your pallas kernel here")


# speedup vs baseline: 8.7884x; 8.7884x over previous
import functools

import jax
import jax.numpy as jnp
from jax import lax
from jax.experimental import pallas as pl
from jax.experimental.pallas import tpu as pltpu
from jax.experimental.pallas import tpu_sc as plsc

N = 10000
D = 128
H = 256
E = 160000

NC = 2
NS = 16
NW = NC * NS
CH = 128
E_PAD = 163840
EPW = E_PAD // NW
NCH = EPW // CH
NA = N + 16
ST = N // NS
DEGW = 16
_SC_PARAMS = pltpu.CompilerParams(use_tc_tiling_on_sc=False)

@functools.cache
def _mesh():
    return plsc.VectorSubcoreMesh(
        core_axis_name="c", subcore_axis_name="s", num_cores=NC, num_subcores=NS
    )


def _sc_degree(dst):
    return _sc_degree_kernel()(dst)


@functools.cache
def _sc_degree_kernel():
    return functools.partial(
        pl.kernel,
        out_type=jax.ShapeDtypeStruct((NC, N, DEGW), jnp.float32),
        mesh=_mesh(),
        scratch_types=[
            pltpu.VMEM((CH,), jnp.int32),
            pltpu.VMEM((CH, DEGW), jnp.float32),
            pltpu.VMEM((ST, DEGW), jnp.float32),
            pltpu.VMEM_SHARED((NA, DEGW), jnp.float32),
        ],
        compiler_params=_SC_PARAMS,
    )(_sc_degree_body)


def _sc_degree_body(dst_hbm, out_hbm, idx_v, ones_v, zero_v, deg_sh):
    cid = lax.axis_index("c")
    sid = lax.axis_index("s")
    wid = cid * NS + sid

    def fill_ones(i, _):
        ones_v[i] = jnp.full((DEGW,), 1.0, jnp.float32)
        return 0

    lax.fori_loop(0, CH, fill_ones, 0)

    def fill_zero(i, _):
        zero_v[i] = jnp.zeros((DEGW,), jnp.float32)
        return 0

    lax.fori_loop(0, ST, fill_zero, 0)
    pltpu.sync_copy(zero_v, deg_sh.at[pl.ds(sid * ST, ST)])

    @pl.when(sid == NS - 1)
    def _():
        pltpu.sync_copy(zero_v.at[pl.ds(0, NA - N)], deg_sh.at[pl.ds(N, NA - N)])

    plsc.subcore_barrier()

    base = wid * EPW

    def body(j, _):
        pltpu.sync_copy(dst_hbm.at[pl.ds(base + j * CH, CH)], idx_v)
        pltpu.sync_copy(ones_v, deg_sh.at[idx_v], add=True)
        return 0

    lax.fori_loop(0, NCH, body, 0)

    plsc.subcore_barrier()
    pltpu.sync_copy(
        deg_sh.at[pl.ds(sid * ST, ST)],
        out_hbm.at[cid, pl.ds(sid * ST, ST)],
    )


def _sc_scatter(y, src, dst):
    return _sc_scatter_kernel()(y, src, dst)


@functools.cache
def _sc_scatter_kernel():
    return functools.partial(
        pl.kernel,
        out_type=jax.ShapeDtypeStruct((NC, N, D), jnp.float32),
        mesh=_mesh(),
        scratch_types=[
            pltpu.VMEM((CH,), jnp.int32),
            pltpu.VMEM((CH,), jnp.int32),
            pltpu.VMEM((CH, D), jnp.float32),
            pltpu.VMEM_SHARED((NA, D), jnp.float32),
            pltpu.SemaphoreType.DMA,
        ],
        compiler_params=_SC_PARAMS,
    )(_sc_scatter_body)


def _sc_scatter_body(
    y_hbm, src_hbm, dst_hbm, out_hbm, si_v, di_v, rows_v, agg_sh, sem,
):
    cid = lax.axis_index("c")
    sid = lax.axis_index("s")
    wid = cid * NS + sid

    def fill_zero(i, _):
        def lane(l, _):
            rows_v[i, pl.ds(l * 16, 16)] = jnp.zeros((16,), jnp.float32)
            return 0

        lax.fori_loop(0, D // 16, lane, 0)
        return 0

    lax.fori_loop(0, CH, fill_zero, 0)
    for z in range(4):
        pltpu.sync_copy(rows_v, agg_sh.at[pl.ds(sid * ST + z * CH, CH)])
    pltpu.sync_copy(
        rows_v.at[pl.ds(0, ST - 4 * CH)],
        agg_sh.at[pl.ds(sid * ST + 4 * CH, ST - 4 * CH)],
    )

    @pl.when(sid == NS - 1)
    def _():
        pltpu.sync_copy(rows_v.at[pl.ds(0, NA - N)], agg_sh.at[pl.ds(N, NA - N)])

    plsc.subcore_barrier()

    base = wid * EPW

    def body(j, _):
        off = base + j * CH
        pltpu.sync_copy(src_hbm.at[pl.ds(off, CH)], si_v)
        pltpu.async_copy(y_hbm.at[si_v], rows_v, sem).wait()
        pltpu.sync_copy(dst_hbm.at[pl.ds(off, CH)], di_v)
        pltpu.sync_copy(rows_v, agg_sh.at[di_v], add=True)
        return 0

    lax.fori_loop(0, NCH, body, 0)

    plsc.subcore_barrier()
    pltpu.sync_copy(
        agg_sh.at[pl.ds(sid * ST, ST)],
        out_hbm.at[cid, pl.ds(sid * ST, ST)],
    )


_BR = 1000
_NB = N // _BR


def _mm_body(x_ref, w_ref, o_ref):
    o_ref[...] = jnp.dot(x_ref[...], w_ref[...], preferred_element_type=jnp.float32, precision=lax.Precision.HIGHEST)


def _tc_matmul(x, Wc):
    return pl.pallas_call(
        _mm_body,
        grid=(_NB,),
        in_specs=[
            pl.BlockSpec((_BR, D), lambda i: (i, 0)),
            pl.BlockSpec((D, D), lambda i: (0, 0)),
        ],
        out_specs=pl.BlockSpec((_BR, D), lambda i: (i, 0)),
        out_shape=jax.ShapeDtypeStruct((N, D), jnp.float32),
    )(x, Wc)


def _scale_body(degp_ref, xw_ref, y_ref, dinv_ref):
    deg = jnp.sum(degp_ref[...], axis=(0, 2)) * (1.0 / DEGW) + 1.0
    dinv = lax.rsqrt(deg)
    dinv_ref[...] = dinv[:, None]
    y_ref[...] = xw_ref[...] * dinv[:, None]


def _tc_scale(degp, xw):
    return pl.pallas_call(
        _scale_body,
        grid=(_NB,),
        in_specs=[
            pl.BlockSpec((NC, _BR, DEGW), lambda i: (0, i, 0)),
            pl.BlockSpec((_BR, D), lambda i: (i, 0)),
        ],
        out_specs=[
            pl.BlockSpec((_BR, D), lambda i: (i, 0)),
            pl.BlockSpec((_BR, 1), lambda i: (i, 0)),
        ],
        out_shape=[
            jax.ShapeDtypeStruct((N, D), jnp.float32),
            jax.ShapeDtypeStruct((N, 1), jnp.float32),
        ],
    )(degp, xw)


def _leaky(v):
    return jnp.where(v >= 0.0, v, 0.01 * v)


def _mlp_body(aggp_ref, y_ref, dinv_ref, x_ref, bc_ref,
              w1_ref, b1_ref, w2_ref, b2_ref, w3_ref, b3_ref,
              conc_ref, tot_ref, acc_ref):
    i = pl.program_id(0)
    agg = aggp_ref[0] + aggp_ref[1] + y_ref[...]
    conv = agg * dinv_ref[...] + bc_ref[...]
    h = jnp.maximum(conv, 0.0) + x_ref[...]
    h1 = _leaky(jnp.dot(h, w1_ref[...], preferred_element_type=jnp.float32, precision=lax.Precision.HIGHEST) + b1_ref[...])
    h2 = _leaky(jnp.dot(h1, w2_ref[...], preferred_element_type=jnp.float32, precision=lax.Precision.HIGHEST) + b2_ref[...])
    z = jnp.dot(h2, w3_ref[...], preferred_element_type=jnp.float32, precision=lax.Precision.HIGHEST) + b3_ref[...]
    c = jnp.maximum(z, 0.0) + jnp.log1p(jnp.exp(-jnp.abs(z)))
    conc_ref[...] = c
    blk = jnp.sum(c)
    acc_ref[0] = jnp.where(i == 0, blk, acc_ref[0] + blk)

    @pl.when(i == _NB - 1)
    def _():
        tot_ref[0, 0] = acc_ref[0]


def _tc_mlp(aggp, y, dinv, x, bc, W1, b1, W2, b2, W3, b3):
    return pl.pallas_call(
        _mlp_body,
        grid=(_NB,),
        in_specs=[
            pl.BlockSpec((NC, _BR, D), lambda i: (0, i, 0)),
            pl.BlockSpec((_BR, D), lambda i: (i, 0)),
            pl.BlockSpec((_BR, 1), lambda i: (i, 0)),
            pl.BlockSpec((_BR, D), lambda i: (i, 0)),
            pl.BlockSpec((1, D), lambda i: (0, 0)),
            pl.BlockSpec((D, H), lambda i: (0, 0)),
            pl.BlockSpec((1, H), lambda i: (0, 0)),
            pl.BlockSpec((H, H), lambda i: (0, 0)),
            pl.BlockSpec((1, H), lambda i: (0, 0)),
            pl.BlockSpec((H, 1), lambda i: (0, 0)),
            pl.BlockSpec((1, 1), lambda i: (0, 0)),
        ],
        out_specs=[
            pl.BlockSpec((_BR, 1), lambda i: (i, 0)),
            pl.BlockSpec(memory_space=pltpu.SMEM),
        ],
        out_shape=[
            jax.ShapeDtypeStruct((N, 1), jnp.float32),
            jax.ShapeDtypeStruct((1, 1), jnp.float32),
        ],
        scratch_shapes=[pltpu.SMEM((1,), jnp.float32)],
    )(aggp, y, dinv, x, bc, W1, b1, W2, b2, W3, b3)


def _norm_body(conc_ref, tot_ref, out_ref):
    out_ref[...] = conc_ref[...] * (1.0 / (tot_ref[0] + 1e-20))


def _tc_norm(conc2, tot):
    return pl.pallas_call(
        _norm_body,
        in_specs=[
            pl.BlockSpec((1, N), lambda: (0, 0)),
            pl.BlockSpec(memory_space=pltpu.SMEM),
        ],
        out_specs=pl.BlockSpec((1, N), lambda: (0, 0)),
        out_shape=jax.ShapeDtypeStruct((1, N), jnp.float32),
    )(conc2, tot)


def kernel(x, edge_index, Wc, bc, W1, b1, W2, b2, W3, b3):
    ei = edge_index.astype(jnp.int32)
    src = jnp.concatenate([ei[0], jnp.zeros((E_PAD - E,), jnp.int32)])
    dst = jnp.concatenate([ei[1], jnp.full((E_PAD - E,), N, jnp.int32)])

    xw = _tc_matmul(x, Wc)
    degp = _sc_degree(dst)
    y, dinv = _tc_scale(degp, xw)
    aggp = _sc_scatter(y, src, dst)
    conc, tot = _tc_mlp(
        aggp, y, dinv, x,
        bc.reshape(1, D), W1, b1.reshape(1, H), W2, b2.reshape(1, H),
        W3, b3.reshape(1, 1),
    )
    action = _tc_norm(conc.reshape(1, N), tot.reshape(1))
    return action


# speedup vs baseline: 10.4541x; 1.1895x over previous
import functools

import jax
import jax.numpy as jnp
from jax import lax
from jax.experimental import pallas as pl
from jax.experimental.pallas import tpu as pltpu
from jax.experimental.pallas import tpu_sc as plsc

N = 10000
D = 128
H = 256
E = 160000

NC = 2
NS = 16
NW = NC * NS
CH = 128
E_PAD = 163840
EPW = E_PAD // NW
NCH = EPW // CH
NA = N + 16
ST = N // NS
DEGW = 16
_SC_PARAMS = pltpu.CompilerParams(use_tc_tiling_on_sc=False)

@functools.cache
def _mesh():
    return plsc.VectorSubcoreMesh(
        core_axis_name="c", subcore_axis_name="s", num_cores=NC, num_subcores=NS
    )


def _sc_degree(dst):
    return _sc_degree_kernel()(dst)


@functools.cache
def _sc_degree_kernel():
    return functools.partial(
        pl.kernel,
        out_type=jax.ShapeDtypeStruct((NC, N, DEGW), jnp.float32),
        mesh=_mesh(),
        scratch_types=[
            pltpu.VMEM((NCH, CH), jnp.int32),
            pltpu.VMEM((CH, DEGW), jnp.float32),
            pltpu.VMEM((ST, DEGW), jnp.float32),
            pltpu.VMEM_SHARED((NA, DEGW), jnp.float32),
            pltpu.SemaphoreType.DMA,
        ],
        compiler_params=_SC_PARAMS,
    )(_sc_degree_body)


def _sc_degree_body(dst_hbm, out_hbm, di_all, ones_v, zero_v, deg_sh, sem):
    cid = lax.axis_index("c")
    sid = lax.axis_index("s")
    wid = cid * NS + sid

    def fill_ones(i, _):
        ones_v[i] = jnp.full((DEGW,), 1.0, jnp.float32)
        return 0

    lax.fori_loop(0, CH, fill_ones, 0)

    def fill_zero(i, _):
        zero_v[i] = jnp.zeros((DEGW,), jnp.float32)
        return 0

    lax.fori_loop(0, ST, fill_zero, 0)
    pltpu.sync_copy(dst_hbm.at[wid], di_all)
    pltpu.sync_copy(zero_v, deg_sh.at[pl.ds(sid * ST, ST)])

    @pl.when(sid == NS - 1)
    def _():
        pltpu.sync_copy(zero_v.at[pl.ds(0, NA - N)], deg_sh.at[pl.ds(N, NA - N)])

    plsc.subcore_barrier()

    def body(j, _):
        pltpu.async_copy(ones_v, deg_sh.at[di_all.at[j]], sem, add=True)
        return 0

    lax.fori_loop(0, NCH, body, 0)

    def drain(j, _):
        pltpu.make_async_copy(ones_v, deg_sh.at[di_all.at[0]], sem).wait()
        return 0

    lax.fori_loop(0, NCH, drain, 0)

    plsc.subcore_barrier()
    pltpu.sync_copy(
        deg_sh.at[pl.ds(sid * ST, ST)],
        out_hbm.at[cid, pl.ds(sid * ST, ST)],
    )


def _sc_scatter(y, src, dst):
    return _sc_scatter_kernel()(y, src, dst)


@functools.cache
def _sc_scatter_kernel():
    return functools.partial(
        pl.kernel,
        out_type=jax.ShapeDtypeStruct((NC, N, D), jnp.float32),
        mesh=_mesh(),
        scratch_types=[
            pltpu.VMEM((NCH, CH), jnp.int32),
            pltpu.VMEM((NCH, CH), jnp.int32),
            pltpu.VMEM((CH, D), jnp.float32),
            pltpu.VMEM((CH, D), jnp.float32),
            pltpu.VMEM_SHARED((NA, D), jnp.float32),
            pltpu.SemaphoreType.DMA,
            pltpu.SemaphoreType.DMA,
            pltpu.SemaphoreType.DMA,
            pltpu.SemaphoreType.DMA,
        ],
        compiler_params=_SC_PARAMS,
    )(_sc_scatter_body)


def _sc_scatter_body(
    y_hbm, src_hbm, dst_hbm, out_hbm,
    si_all, di_all, rows0_v, rows1_v, agg_sh, gsem0, gsem1, ssem0, ssem1,
):
    cid = lax.axis_index("c")
    sid = lax.axis_index("s")
    wid = cid * NS + sid

    def fill_zero(i, _):
        def lane(l, _):
            rows0_v[i, pl.ds(l * 16, 16)] = jnp.zeros((16,), jnp.float32)
            return 0

        lax.fori_loop(0, D // 16, lane, 0)
        return 0

    lax.fori_loop(0, CH, fill_zero, 0)
    pltpu.sync_copy(src_hbm.at[wid], si_all)
    pltpu.sync_copy(dst_hbm.at[wid], di_all)
    for z in range(4):
        pltpu.sync_copy(rows0_v, agg_sh.at[pl.ds(sid * ST + z * CH, CH)])
    pltpu.sync_copy(
        rows0_v.at[pl.ds(0, ST - 4 * CH)],
        agg_sh.at[pl.ds(sid * ST + 4 * CH, ST - 4 * CH)],
    )

    @pl.when(sid == NS - 1)
    def _():
        pltpu.sync_copy(rows0_v.at[pl.ds(0, NA - N)], agg_sh.at[pl.ds(N, NA - N)])

    plsc.subcore_barrier()

    def g(j, buf, sem):
        pltpu.async_copy(y_hbm.at[si_all.at[j]], buf, sem)

    def gw(buf, sem):
        pltpu.make_async_copy(y_hbm.at[si_all.at[0]], buf, sem).wait()

    def s(j, buf, sem):
        pltpu.async_copy(buf, agg_sh.at[di_all.at[j]], sem, add=True)

    def sw(buf, sem):
        pltpu.make_async_copy(buf, agg_sh.at[di_all.at[0]], sem).wait()

    g(0, rows0_v, gsem0)
    g(1, rows1_v, gsem1)

    def body(p, _):
        j0 = 2 * p
        j1 = j0 + 1
        gw(rows0_v, gsem0)
        s(j0, rows0_v, ssem0)
        gw(rows1_v, gsem1)
        s(j1, rows1_v, ssem1)
        sw(rows0_v, ssem0)
        g(lax.rem(j0 + 2, NCH), rows0_v, gsem0)
        sw(rows1_v, ssem1)
        g(lax.rem(j1 + 2, NCH), rows1_v, gsem1)
        return 0

    lax.fori_loop(0, NCH // 2, body, 0)
    gw(rows0_v, gsem0)
    gw(rows1_v, gsem1)

    plsc.subcore_barrier()
    pltpu.sync_copy(
        agg_sh.at[pl.ds(sid * ST, ST)],
        out_hbm.at[cid, pl.ds(sid * ST, ST)],
    )


_BR = 1000
_NB = N // _BR


def _mm_body(x_ref, w_ref, o_ref):
    o_ref[...] = jnp.dot(x_ref[...], w_ref[...], preferred_element_type=jnp.float32, precision=lax.Precision.HIGHEST)


def _tc_matmul(x, Wc):
    return pl.pallas_call(
        _mm_body,
        grid=(_NB,),
        in_specs=[
            pl.BlockSpec((_BR, D), lambda i: (i, 0)),
            pl.BlockSpec((D, D), lambda i: (0, 0)),
        ],
        out_specs=pl.BlockSpec((_BR, D), lambda i: (i, 0)),
        out_shape=jax.ShapeDtypeStruct((N, D), jnp.float32),
    )(x, Wc)


def _scale_body(degp_ref, xw_ref, y_ref, dinv_ref):
    deg = jnp.sum(degp_ref[...], axis=(0, 2)) * (1.0 / DEGW) + 1.0
    dinv = lax.rsqrt(deg)
    dinv_ref[...] = dinv[:, None]
    y_ref[...] = xw_ref[...] * dinv[:, None]


def _tc_scale(degp, xw):
    return pl.pallas_call(
        _scale_body,
        grid=(_NB,),
        in_specs=[
            pl.BlockSpec((NC, _BR, DEGW), lambda i: (0, i, 0)),
            pl.BlockSpec((_BR, D), lambda i: (i, 0)),
        ],
        out_specs=[
            pl.BlockSpec((_BR, D), lambda i: (i, 0)),
            pl.BlockSpec((_BR, 1), lambda i: (i, 0)),
        ],
        out_shape=[
            jax.ShapeDtypeStruct((N, D), jnp.float32),
            jax.ShapeDtypeStruct((N, 1), jnp.float32),
        ],
    )(degp, xw)


def _leaky(v):
    return jnp.where(v >= 0.0, v, 0.01 * v)


def _mlp_body(aggp_ref, y_ref, dinv_ref, x_ref, bc_ref,
              w1_ref, b1_ref, w2_ref, b2_ref, w3_ref, b3_ref,
              conc_ref, tot_ref, acc_ref):
    i = pl.program_id(0)
    agg = aggp_ref[0] + aggp_ref[1] + y_ref[...]
    conv = agg * dinv_ref[...] + bc_ref[...]
    h = jnp.maximum(conv, 0.0) + x_ref[...]
    h1 = _leaky(jnp.dot(h, w1_ref[...], preferred_element_type=jnp.float32, precision=lax.Precision.HIGHEST) + b1_ref[...])
    h2 = _leaky(jnp.dot(h1, w2_ref[...], preferred_element_type=jnp.float32, precision=lax.Precision.HIGHEST) + b2_ref[...])
    z = jnp.dot(h2, w3_ref[...], preferred_element_type=jnp.float32, precision=lax.Precision.HIGHEST) + b3_ref[...]
    c = jnp.maximum(z, 0.0) + jnp.log1p(jnp.exp(-jnp.abs(z)))
    conc_ref[...] = c
    blk = jnp.sum(c)
    acc_ref[0] = jnp.where(i == 0, blk, acc_ref[0] + blk)

    @pl.when(i == _NB - 1)
    def _():
        tot_ref[0, 0] = acc_ref[0]


def _tc_mlp(aggp, y, dinv, x, bc, W1, b1, W2, b2, W3, b3):
    return pl.pallas_call(
        _mlp_body,
        grid=(_NB,),
        in_specs=[
            pl.BlockSpec((NC, _BR, D), lambda i: (0, i, 0)),
            pl.BlockSpec((_BR, D), lambda i: (i, 0)),
            pl.BlockSpec((_BR, 1), lambda i: (i, 0)),
            pl.BlockSpec((_BR, D), lambda i: (i, 0)),
            pl.BlockSpec((1, D), lambda i: (0, 0)),
            pl.BlockSpec((D, H), lambda i: (0, 0)),
            pl.BlockSpec((1, H), lambda i: (0, 0)),
            pl.BlockSpec((H, H), lambda i: (0, 0)),
            pl.BlockSpec((1, H), lambda i: (0, 0)),
            pl.BlockSpec((H, 1), lambda i: (0, 0)),
            pl.BlockSpec((1, 1), lambda i: (0, 0)),
        ],
        out_specs=[
            pl.BlockSpec((_BR, 1), lambda i: (i, 0)),
            pl.BlockSpec(memory_space=pltpu.SMEM),
        ],
        out_shape=[
            jax.ShapeDtypeStruct((N, 1), jnp.float32),
            jax.ShapeDtypeStruct((1, 1), jnp.float32),
        ],
        scratch_shapes=[pltpu.SMEM((1,), jnp.float32)],
    )(aggp, y, dinv, x, bc, W1, b1, W2, b2, W3, b3)


def _norm_body(conc_ref, tot_ref, out_ref):
    out_ref[...] = conc_ref[...] * (1.0 / (tot_ref[0] + 1e-20))


def _tc_norm(conc2, tot):
    return pl.pallas_call(
        _norm_body,
        in_specs=[
            pl.BlockSpec((1, N), lambda: (0, 0)),
            pl.BlockSpec(memory_space=pltpu.SMEM),
        ],
        out_specs=pl.BlockSpec((1, N), lambda: (0, 0)),
        out_shape=jax.ShapeDtypeStruct((1, N), jnp.float32),
    )(conc2, tot)


def kernel(x, edge_index, Wc, bc, W1, b1, W2, b2, W3, b3):
    ei = edge_index.astype(jnp.int32)
    src = jnp.concatenate(
        [ei[0], jnp.zeros((E_PAD - E,), jnp.int32)]).reshape(NW, NCH, CH)
    dst = jnp.concatenate(
        [ei[1], jnp.full((E_PAD - E,), N, jnp.int32)]).reshape(NW, NCH, CH)

    xw = _tc_matmul(x, Wc)
    degp = _sc_degree(dst)
    y, dinv = _tc_scale(degp, xw)
    aggp = _sc_scatter(y, src, dst)
    conc, tot = _tc_mlp(
        aggp, y, dinv, x,
        bc.reshape(1, D), W1, b1.reshape(1, H), W2, b2.reshape(1, H),
        W3, b3.reshape(1, 1),
    )
    action = _tc_norm(conc.reshape(1, N), tot.reshape(1))
    return action
